# baseline (speedup 1.0000x reference)
import functools
import math

import jax
import jax.numpy as jnp
from jax.experimental import pallas as pl
from jax.experimental.pallas import tpu as pltpu


def _round_up(x: int, m: int) -> int:
    return ((x + m - 1) // m) * m


_CHUNK = 128


def _adapter_body(gamma_ref, x_ref, w1_ref, b1_ref, w2_ref, b2_ref, out_ref):
    tm = x_ref.shape[0]
    gamma = gamma_ref[0, 0]
    for r in range(0, tm, _CHUNK):
        sl = pl.ds(r, _CHUNK)
        x = x_ref[sl, :]
        h = jnp.dot(x.astype(jnp.bfloat16), w1_ref[...],
                    preferred_element_type=jnp.float32) + b1_ref[...]
        h = 0.5 * h * (1.0 + jax.lax.erf(h * jnp.float32(1.0 / math.sqrt(2.0))))
        y = jnp.dot(h.astype(jnp.bfloat16), w2_ref[...],
                    preferred_element_type=jnp.float32) + b2_ref[...]
        out = x + gamma * y
        out_ref[sl, :, :] = out.reshape(_CHUNK, 8, out_ref.shape[2])


@functools.partial(jax.jit, static_argnames=("tm",))
def _adapter(x, gamma_arr, w1_bf, b1_f32, w2_bf, b2_f32, *, tm):
    n, d = x.shape
    n_pad = _round_up(n, tm)
    x_p = jnp.pad(x, ((0, n_pad - n), (0, 0))) if n_pad != n else x

    cost = pl.CostEstimate(
        flops=4 * n_pad * d * d,
        transcendentals=n_pad * d,
        bytes_accessed=8 * n_pad * d + 4 * d * d + 8 * d,
    )

    out = pl.pallas_call(
        _adapter_body,
        out_shape=jax.ShapeDtypeStruct((n_pad, 8, d // 8), jnp.float32),
        grid=(n_pad // tm,),
        in_specs=[
            pl.BlockSpec(memory_space=pltpu.MemorySpace.SMEM),
            pl.BlockSpec((tm, d), lambda i: (i, 0)),
            pl.BlockSpec((d, d), lambda i: (0, 0)),
            pl.BlockSpec((1, d), lambda i: (0, 0)),
            pl.BlockSpec((d, d), lambda i: (0, 0)),
            pl.BlockSpec((1, d), lambda i: (0, 0)),
        ],
        out_specs=pl.BlockSpec((tm, 8, d // 8), lambda i: (i, 0, 0)),
        compiler_params=pltpu.CompilerParams(
            dimension_semantics=("parallel",),
            vmem_limit_bytes=100 << 20,
        ),
        cost_estimate=cost,
    )(gamma_arr, x_p, w1_bf, b1_f32, w2_bf, b2_f32)

    if n_pad != n:
        out = out[:n]
    return out.reshape(n, 1, d)


def kernel(x, gamma, w1_bf, b1_f32, w2_bf, b2_f32):
    gamma_arr = jnp.asarray(gamma, jnp.float32).reshape(1, 1)
    return _adapter(x, gamma_arr, jnp.asarray(w1_bf, jnp.bfloat16),
                    jnp.asarray(b1_f32, jnp.float32).reshape(1, -1),
                    jnp.asarray(w2_bf, jnp.bfloat16),
                    jnp.asarray(b2_f32, jnp.float32).reshape(1, -1), tm=1024)


# speedup vs baseline: 1.1726x; 1.1726x over previous
import functools
import math

import jax
import jax.numpy as jnp
from jax.experimental import pallas as pl
from jax.experimental.pallas import tpu as pltpu


def _round_up(x: int, m: int) -> int:
    return ((x + m - 1) // m) * m


def _adapter_body(gamma_ref, x_ref, w1_ref, b1_ref, w2_ref, b2_ref, out_ref):
    tm = x_ref.shape[0]
    x = x_ref[...]
    h = jnp.dot(x.astype(jnp.bfloat16), w1_ref[...],
                preferred_element_type=jnp.float32) + b1_ref[...]
    h = 0.5 * h * (1.0 + jax.lax.erf(h * jnp.float32(1.0 / math.sqrt(2.0))))
    y = jnp.dot(h.astype(jnp.bfloat16), w2_ref[...],
                preferred_element_type=jnp.float32) + b2_ref[...]
    out = x + gamma_ref[0, 0] * y
    out_ref[...] = out.reshape(tm, 8, out_ref.shape[2])


@functools.partial(jax.jit, static_argnames=("tm",))
def _adapter(x, gamma_arr, w1_bf, b1_f32, w2_bf, b2_f32, *, tm):
    n, d = x.shape
    n_pad = _round_up(n, tm)
    x_p = jnp.pad(x, ((0, n_pad - n), (0, 0))) if n_pad != n else x

    cost = pl.CostEstimate(
        flops=4 * n_pad * d * d,
        transcendentals=n_pad * d,
        bytes_accessed=8 * n_pad * d + 4 * d * d + 8 * d,
    )

    out = pl.pallas_call(
        _adapter_body,
        out_shape=jax.ShapeDtypeStruct((n_pad, 8, d // 8), jnp.float32),
        grid=(n_pad // tm,),
        in_specs=[
            pl.BlockSpec(memory_space=pltpu.MemorySpace.SMEM),
            pl.BlockSpec((tm, d), lambda i: (i, 0)),
            pl.BlockSpec((d, d), lambda i: (0, 0),
                         pipeline_mode=pl.Buffered(1)),
            pl.BlockSpec((1, d), lambda i: (0, 0),
                         pipeline_mode=pl.Buffered(1)),
            pl.BlockSpec((d, d), lambda i: (0, 0),
                         pipeline_mode=pl.Buffered(1)),
            pl.BlockSpec((1, d), lambda i: (0, 0),
                         pipeline_mode=pl.Buffered(1)),
        ],
        out_specs=pl.BlockSpec((tm, 8, d // 8), lambda i: (i, 0, 0)),
        compiler_params=pltpu.CompilerParams(
            dimension_semantics=("parallel",),
            vmem_limit_bytes=100 << 20,
        ),
        cost_estimate=cost,
    )(gamma_arr, x_p, w1_bf, b1_f32, w2_bf, b2_f32)

    if n_pad != n:
        out = out[:n]
    return out.reshape(n, 1, d)


def kernel(x, gamma, w1_bf, b1_f32, w2_bf, b2_f32):
    gamma_arr = jnp.asarray(gamma, jnp.float32).reshape(1, 1)
    return _adapter(x, gamma_arr, jnp.asarray(w1_bf, jnp.bfloat16),
                    jnp.asarray(b1_f32, jnp.float32).reshape(1, -1),
                    jnp.asarray(w2_bf, jnp.bfloat16),
                    jnp.asarray(b2_f32, jnp.float32).reshape(1, -1), tm=1024)
